# baseline (speedup 1.0000x reference)
import functools

import jax
import jax.numpy as jnp
from jax import lax
from jax.experimental import pallas as pl
from jax.experimental.pallas import tpu as pltpu
from jax.experimental.pallas import tpu_sc as plsc

NW = 32
C = 32


NB = 2


def _emb_body(idx_hbm, table_hbm, out_hbm, idx_v, shared, *rest):
    rows = rest[:NB]
    gsems = rest[NB:2 * NB]
    wsems = rest[2 * NB:3 * NB]
    csems = rest[2 * NB:3 * NB]
    nch = idx_hbm.shape[1]
    sid = lax.axis_index("s")
    wid = sid * 2 + lax.axis_index("c")
    slabs = [shared.at[sid * NB + i] for i in range(NB)]
    per_w = nch * C
    base = wid * per_w
    pltpu.sync_copy(idx_hbm.at[wid], idx_v)

    def out_slab(j):
        return out_hbm.at[pl.ds(pl.multiple_of(base + j * C, 8), C)]

    def start_gather(j, i):
        pltpu.async_copy(table_hbm.at[idx_v.at[j]], rows[i], gsems[i])

    def wait_gather(i):
        pltpu.make_async_copy(table_hbm.at[idx_v.at[0]], rows[i],
                              gsems[i]).wait()

    def start_write(j, i):
        pltpu.sync_copy(rows[i], slabs[i])
        pltpu.async_copy(slabs[i], out_slab(j), wsems[i])

    def wait_write(i):
        pltpu.make_async_copy(slabs[i], out_slab(0), wsems[i]).wait()

    for i in range(NB):
        start_gather(i, i)

    def round_(kk, carry):
        j0 = kk * NB
        for i in range(NB):
            wait_gather(i)
            start_write(j0 + i, i)
        for i in range(NB):
            wait_write(i)
            start_gather(j0 + NB + i, i)
        return carry

    lax.fori_loop(0, nch // NB - 1, round_, 0)

    j0 = nch - NB
    for i in range(NB):
        wait_gather(i)
        start_write(j0 + i, i)
    for i in range(NB):
        wait_write(i)


def kernel(input_ids, embed_table):
    b, s = input_ids.shape
    v, d = embed_table.shape
    ntok = b * s
    per_w = ntok // NW
    nch = per_w // C
    ids = input_ids.reshape(-1).astype(jnp.int32).reshape(NW, nch, C)

    run = functools.partial(
        pl.kernel,
        mesh=plsc.VectorSubcoreMesh(core_axis_name="c", subcore_axis_name="s"),
        out_type=jax.ShapeDtypeStruct((ntok, d), jnp.float32),
        scratch_types=(
            [pltpu.VMEM((nch, C), jnp.int32)]
            + [pltpu.VMEM_SHARED((16 * NB, C, d), jnp.float32)]
            + [pltpu.VMEM((C, d), jnp.float32) for _ in range(NB)]
            + [pltpu.SemaphoreType.DMA for _ in range(2 * NB)]
        ),
    )(_emb_body)

    out = run(ids, embed_table)
    return out.reshape(b, s, d)


# speedup vs baseline: 1.0125x; 1.0125x over previous
import functools

import jax
import jax.numpy as jnp
from jax import lax
from jax.experimental import pallas as pl
from jax.experimental.pallas import tpu as pltpu
from jax.experimental.pallas import tpu_sc as plsc

NW = 32
C = 64


NB = 2


def _emb_body(idx_hbm, table_hbm, out_hbm, idx_v, *rest):
    rows = rest[:NB]
    gsems = rest[NB:2 * NB]
    wsems = rest[2 * NB:3 * NB]
    nch = idx_hbm.shape[1]
    wid = lax.axis_index("s") * 2 + lax.axis_index("c")
    per_w = nch * C
    base = wid * per_w
    pltpu.sync_copy(idx_hbm.at[wid], idx_v)

    def out_slab(j):
        return out_hbm.at[pl.ds(pl.multiple_of(base + j * C, 8), C)]

    def start_gather(j, i):
        pltpu.async_copy(table_hbm.at[idx_v.at[j]], rows[i], gsems[i])

    def wait_gather(i):
        pltpu.make_async_copy(table_hbm.at[idx_v.at[0]], rows[i],
                              gsems[i]).wait()

    def start_write(j, i):
        pltpu.async_copy(rows[i], out_slab(j), wsems[i])

    def wait_write(i):
        pltpu.make_async_copy(rows[i], out_slab(0), wsems[i]).wait()

    for i in range(NB):
        start_gather(i, i)

    def round_(kk, carry):
        j0 = kk * NB
        for i in range(NB):
            wait_gather(i)
            start_write(j0 + i, i)
        for i in range(NB):
            wait_write(i)
            start_gather(j0 + NB + i, i)
        return carry

    lax.fori_loop(0, nch // NB - 1, round_, 0)

    j0 = nch - NB
    for i in range(NB):
        wait_gather(i)
        start_write(j0 + i, i)
    for i in range(NB):
        wait_write(i)


def kernel(input_ids, embed_table):
    b, s = input_ids.shape
    v, d = embed_table.shape
    ntok = b * s
    per_w = ntok // NW
    nch = per_w // C
    ids = input_ids.reshape(-1).astype(jnp.int32).reshape(NW, nch, C)

    run = functools.partial(
        pl.kernel,
        mesh=plsc.VectorSubcoreMesh(core_axis_name="c", subcore_axis_name="s"),
        out_type=jax.ShapeDtypeStruct((ntok, d), jnp.float32),
        scratch_types=(
            [pltpu.VMEM((nch, C), jnp.int32)]
            + [pltpu.VMEM((C, d), jnp.float32) for _ in range(NB)]
            + [pltpu.SemaphoreType.DMA for _ in range(2 * NB)]
        ),
    )(_emb_body)

    out = run(ids, embed_table)
    return out.reshape(b, s, d)
